# baseline (speedup 1.0000x reference)
import functools

import jax
import jax.numpy as jnp
from jax import lax
from jax.experimental import pallas as pl
from jax.experimental.pallas import tpu as pltpu
from jax.experimental.pallas import tpu_sc as plsc

EMBED = 64
B_TOTAL = 4096 * 200
NC, NS = 2, 16
NW = NC * NS
PER_W = B_TOTAL // NW
SUB = 128
ROWS_W = PER_W // SUB
GROUP = 256
NSUB = GROUP // SUB
NGROUP = PER_W // GROUP
NBUF = 4
NROUND = NGROUP // NBUF

_mesh = plsc.VectorSubcoreMesh(core_axis_name="c", subcore_axis_name="s")


@functools.partial(
    pl.kernel,
    mesh=_mesh,
    out_type=jax.ShapeDtypeStruct((B_TOTAL, EMBED), jnp.float32),
    scratch_types=[
        pltpu.VMEM((ROWS_W, SUB), jnp.int32),
        pltpu.VMEM((NBUF, GROUP, EMBED), jnp.float32),
        pltpu.SemaphoreType.DMA((NBUF,)),
        pltpu.SemaphoreType.DMA((NBUF,)),
    ],
    compiler_params=pltpu.CompilerParams(use_tc_tiling_on_sc=False),
)
def _embed_lookup(token_hbm, table_hbm, out_hbm, idx_v, rows_v, gsem, osem):
    wid = lax.axis_index("s") * NC + lax.axis_index("c")
    row_base = wid * ROWS_W
    out_base = wid * PER_W

    pltpu.sync_copy(token_hbm.at[pl.ds(row_base, ROWS_W)], idx_v)

    def start_gather(g, b):
        for j in range(NSUB):
            pltpu.async_copy(
                table_hbm.at[idx_v.at[g * NSUB + j]],
                rows_v.at[b, pl.ds(j * SUB, SUB)],
                gsem.at[b],
            )

    def wait_gather(b):
        pltpu.make_async_copy(
            table_hbm.at[pl.ds(0, GROUP)], rows_v.at[b], gsem.at[b]
        ).wait()

    for b in range(NBUF):
        start_gather(b, b)

    def round_body(r, carry):
        g0 = r * NBUF
        handles = []
        for b in range(NBUF):
            g = g0 + b
            wait_gather(b)
            off = pl.multiple_of(out_base + g * GROUP, GROUP)
            h = pltpu.async_copy(
                rows_v.at[b], out_hbm.at[pl.ds(off, GROUP)], osem.at[b]
            )
            handles.append((h, g, b))
        for h, g, b in handles:
            h.wait()

            @pl.when(g + NBUF < NGROUP)
            def _():
                start_gather(g + NBUF, b)

        return carry

    lax.fori_loop(0, NROUND, round_body, 0)


def kernel(token, embed):
    shape = token.shape
    tok = token.reshape(B_TOTAL // SUB, SUB).astype(jnp.int32)
    out = _embed_lookup(tok, embed)
    return out.reshape(*shape, EMBED)


# speedup vs baseline: 1.0020x; 1.0020x over previous
import functools

import jax
import jax.numpy as jnp
from jax import lax
from jax.experimental import pallas as pl
from jax.experimental.pallas import tpu as pltpu
from jax.experimental.pallas import tpu_sc as plsc

EMBED = 64
B_TOTAL = 4096 * 200
NC, NS = 2, 16
NW = NC * NS
PER_W = B_TOTAL // NW
SUB = 128
ROWS_W = PER_W // SUB
GROUP = 128
NSUB = GROUP // SUB
NGROUP = PER_W // GROUP
NBUF = 8
KPRE = 4
NROUND = NGROUP // NBUF

_mesh = plsc.VectorSubcoreMesh(core_axis_name="c", subcore_axis_name="s")


@functools.partial(
    pl.kernel,
    mesh=_mesh,
    out_type=jax.ShapeDtypeStruct((B_TOTAL, EMBED), jnp.float32),
    scratch_types=[
        pltpu.VMEM((ROWS_W, SUB), jnp.int32),
        pltpu.VMEM((NBUF, GROUP, EMBED), jnp.float32),
        pltpu.SemaphoreType.DMA((NBUF,)),
        pltpu.SemaphoreType.DMA((NBUF,)),
    ],
    compiler_params=pltpu.CompilerParams(use_tc_tiling_on_sc=False),
)
def _embed_lookup(token_hbm, table_hbm, out_hbm, idx_v, rows_v, gsem, osem):
    wid = lax.axis_index("s") * NC + lax.axis_index("c")
    row_base = wid * ROWS_W
    out_base = wid * PER_W

    pltpu.sync_copy(token_hbm.at[pl.ds(row_base, ROWS_W)], idx_v)

    def start_gather(g, b):
        for j in range(NSUB):
            pltpu.async_copy(
                table_hbm.at[idx_v.at[g * NSUB + j]],
                rows_v.at[b, pl.ds(j * SUB, SUB)],
                gsem.at[b],
            )

    def wait_gather(b):
        pltpu.make_async_copy(
            table_hbm.at[pl.ds(0, GROUP)], rows_v.at[b], gsem.at[b]
        ).wait()

    def start_out(g, b):
        off = pl.multiple_of(out_base + g * GROUP, GROUP)
        pltpu.async_copy(rows_v.at[b], out_hbm.at[pl.ds(off, GROUP)], osem.at[b])

    def wait_out(g, b):
        off = pl.multiple_of(out_base + g * GROUP, GROUP)
        pltpu.make_async_copy(
            rows_v.at[b], out_hbm.at[pl.ds(off, GROUP)], osem.at[b]
        ).wait()

    for b in range(NBUF):
        start_gather(b, b)

    def round_body(r, carry):
        g0 = r * NBUF
        for j in range(NBUF):
            g = g0 + j
            t = g + KPRE
            tb = (j + KPRE) % NBUF

            @pl.when(jnp.logical_and(t >= NBUF, t < NGROUP))
            def _():
                wait_out(t - NBUF, tb)
                start_gather(t, tb)

            wait_gather(j)
            start_out(g, j)
        return carry

    lax.fori_loop(0, NROUND, round_body, 0)

    for i in range(NBUF - KPRE):
        s = NGROUP - (NBUF - KPRE) + i
        wait_out(s, s % NBUF)


def kernel(token, embed):
    shape = token.shape
    tok = token.reshape(B_TOTAL // SUB, SUB).astype(jnp.int32)
    out = _embed_lookup(tok, embed)
    return out.reshape(*shape, EMBED)
